# baseline (speedup 1.0000x reference)
import functools

import jax
import jax.numpy as jnp
import numpy as np
from jax.experimental import pallas as pl
from jax.experimental.pallas import tpu as pltpu

_BINS = 10
_TOT = 16384 * 1000
_ROWS = 16000
_COLS = 1024
_BLOCK_R = 1000
_EDGES = [np.float32(i / _BINS) for i in range(1, _BINS)] + [np.float32(1.0 + 1e-6)]


def _ghm_body(p_ref, t_ref, out_ref, acc_ref):
    k = pl.program_id(0)

    @pl.when(k == 0)
    def _init():
        for j in range(22):
            acc_ref[j] = jnp.float32(0.0)

    p = p_ref[...]
    t = t_ref[...]
    g = jnp.abs(p - t)
    logp = jnp.maximum(jnp.log(p), -100.0)
    log1mp = jnp.maximum(jnp.log(1.0 - p), -100.0)
    bce = -(t * logp + (1.0 - t) * log1mp)

    acc_ref[0] = acc_ref[0] + jnp.float32(_BLOCK_R * _COLS)
    acc_ref[11] = acc_ref[11] + jnp.sum(bce)
    for i in range(_BINS):
        m = g >= _EDGES[i]
        acc_ref[1 + i] = acc_ref[1 + i] + jnp.sum(m.astype(jnp.float32))
        acc_ref[12 + i] = acc_ref[12 + i] + jnp.sum(jnp.where(m, bce, 0.0))

    @pl.when(k == pl.num_programs(0) - 1)
    def _finish():
        tot = jnp.float32(_TOT)
        num = jnp.float32(0.0)
        nbins = jnp.float32(0.0)
        for b in range(_BINS):
            c = acc_ref[b] - acc_ref[b + 1]
            s = acc_ref[11 + b] - acc_ref[12 + b]
            ema = 0.25 * c
            w = jnp.where(c > 0, tot / jnp.maximum(ema, jnp.float32(1e-12)),
                          jnp.float32(0.0))
            num = num + w * s
            nbins = nbins + (c > 0).astype(jnp.float32)
        out_ref[0, 0] = num / (jnp.maximum(nbins, 1.0) * tot)


@functools.partial(jax.jit)
def kernel(inputs, targets):
    p = inputs.reshape(_ROWS, _COLS)
    t = targets.reshape(_ROWS, _COLS)
    grid = _ROWS // _BLOCK_R
    out = pl.pallas_call(
        _ghm_body,
        grid=(grid,),
        in_specs=[
            pl.BlockSpec((_BLOCK_R, _COLS), lambda i: (i, 0)),
            pl.BlockSpec((_BLOCK_R, _COLS), lambda i: (i, 0)),
        ],
        out_specs=pl.BlockSpec(memory_space=pltpu.SMEM),
        out_shape=jax.ShapeDtypeStruct((1, 1), jnp.float32),
        scratch_shapes=[pltpu.SMEM((22,), jnp.float32)],
    )(p, t)
    return out[0, 0]
